# baseline (speedup 1.0000x reference)
import jax
import jax.numpy as jnp
from jax.experimental import pallas as pl
from jax.experimental.pallas import tpu as pltpu

_N = 4096
_TM = 512
_NT = _N // _TM
_H = 32
_Z = 16
_C = 7


def _adj_new_tile(mean_tile, mean_full, max_val, row_base):
    l = jax.lax.dot_general(mean_tile, mean_full, (((1,), (1,)), ((), ())))
    s = jnp.round(l / max_val)
    rows = jax.lax.broadcasted_iota(jnp.int32, s.shape, 0) + row_base
    cols = jax.lax.broadcasted_iota(jnp.int32, s.shape, 1)
    return jnp.where(rows == cols, 1.0, s)


def _mega_kernel(adj_ref, feats_ref, wb_ref, wm_ref, wn0_ref, b0_ref,
                 wn1_ref, b1_ref,
                 nc_ref, logits_ref,
                 fb_s, f2_s, t2_s, mean_s, mx_s, d_s, v_s):
    p = pl.program_id(0)
    i = pl.program_id(1)
    rows = pl.ds(i * _TM, _TM)

    @pl.when((p == 0) & (i == 0))
    def _():
        f = feats_ref[...]
        fb_s[...] = jnp.dot(f, wb_ref[...])
        f2_s[...] = jnp.dot(f, wn0_ref[...])

    @pl.when(p == 0)
    def _():
        t2_s[rows, :] = jnp.dot(jnp.dot(adj_ref[...], fb_s[...]),
                                wm_ref[...])

    @pl.when(p == 1)
    def _():
        mean_s[rows, :] = jax.nn.relu(jnp.dot(adj_ref[...], t2_s[...]))

    @pl.when(p == 2)
    def _():
        l = jax.lax.dot_general(mean_s[rows, :], mean_s[...],
                                (((1,), (1,)), ((), ())))
        logits_ref[...] = l
        tile_max = jnp.max(l).reshape(1, 1)

        @pl.when(i == 0)
        def _():
            mx_s[...] = tile_max

        @pl.when(i != 0)
        def _():
            mx_s[...] = jnp.maximum(mx_s[...], tile_max)

    @pl.when(p == 3)
    def _():
        a = _adj_new_tile(mean_s[rows, :], mean_s[...], mx_s[...], i * _TM)
        d_s[rows, :] = jnp.power(jnp.sum(a, axis=1, keepdims=True), -0.5)

    @pl.when(p == 4)
    def _():
        a = _adj_new_tile(mean_s[rows, :], mean_s[...], mx_s[...], i * _TM)
        xd = d_s[...] * f2_s[...]
        h = jax.nn.relu(d_s[rows, :] * jnp.dot(a, xd) + b0_ref[...])
        yd = d_s[rows, :] * jnp.dot(h, wn1_ref[...])
        v_part = jax.lax.dot_general(a, yd, (((0,), (0,)), ((), ())))

        @pl.when(i == 0)
        def _():
            v_s[...] = v_part

        @pl.when(i != 0)
        def _():
            v_s[...] = v_s[...] + v_part

        @pl.when(i == _NT - 1)
        def _():
            nc_ref[...] = d_s[...] * v_s[...] + b1_ref[...]


def _const(shape):
    return pl.BlockSpec(shape, lambda p, i: (0,) * len(shape))


def kernel(adj, adj_orig, features, W_base, W_mean, W_nc0, b_nc0, W_nc1, b_nc1):
    del adj_orig
    f32 = jnp.float32
    b0 = b_nc0.reshape(1, _H)
    b1 = b_nc1.reshape(1, _C)

    adj_spec = pl.BlockSpec((_TM, _N),
                            lambda p, i: (jnp.where(p <= 1, i, _NT - 1), 0))
    logits_spec = pl.BlockSpec(
        (_TM, _N),
        lambda p, i: (jnp.where(p < 2, 0, jnp.where(p == 2, i, _NT - 1)), 0))

    nco, adj_logits = pl.pallas_call(
        _mega_kernel,
        grid=(5, _NT),
        in_specs=[adj_spec, _const((_N, 128)), _const((128, _H)),
                  _const((_H, _Z)), _const((128, _H)), _const((1, _H)),
                  _const((_H, _C)), _const((1, _C))],
        out_specs=[_const((_N, _C)), logits_spec],
        out_shape=[jax.ShapeDtypeStruct((_N, _C), f32),
                   jax.ShapeDtypeStruct((_N, _N), f32)],
        scratch_shapes=[pltpu.VMEM((_N, _H), f32), pltpu.VMEM((_N, _H), f32),
                        pltpu.VMEM((_N, _Z), f32), pltpu.VMEM((_N, _Z), f32),
                        pltpu.VMEM((1, 1), f32), pltpu.VMEM((_N, 1), f32),
                        pltpu.VMEM((_N, _C), f32)],
        compiler_params=pltpu.CompilerParams(
            vmem_limit_bytes=60 * 1024 * 1024),
    )(adj, features, W_base, W_mean, W_nc0, b0, W_nc1, b1)

    return (nco, adj_logits)


# speedup vs baseline: 1.0835x; 1.0835x over previous
import jax
import jax.numpy as jnp
from jax.experimental import pallas as pl
from jax.experimental.pallas import tpu as pltpu

_N = 4096
_TM = 512
_NT = _N // _TM
_H = 32
_Z = 16
_C = 7


def _adj_new_tile(mean_tile, mean_full, max_val, row_base):
    l = jax.lax.dot_general(mean_tile, mean_full, (((1,), (1,)), ((), ())))
    s = jnp.round(l / max_val)
    rows = jax.lax.broadcasted_iota(jnp.int32, s.shape, 0) + row_base
    cols = jax.lax.broadcasted_iota(jnp.int32, s.shape, 1)
    return jnp.where(rows == cols, 1.0, s)


def _mega_kernel(adj_ref, feats_ref, wb_ref, wm_ref, wn0_ref, b0_ref,
                 wn1_ref, b1_ref,
                 nc_ref, logits_ref,
                 fb_s, f2_s, t2_s, mean_s, mx_s, d_s, v_s):
    p = pl.program_id(0)
    i = pl.program_id(1)
    rows = pl.ds(i * _TM, _TM)

    @pl.when((p == 0) & (i == 0))
    def _():
        f = feats_ref[...]
        fb_s[...] = jnp.dot(f, wb_ref[...])
        f2_s[...] = jnp.dot(f, wn0_ref[...])

    @pl.when(p == 0)
    def _():
        t2_s[rows, :] = jnp.dot(jnp.dot(adj_ref[...], fb_s[...]),
                                wm_ref[...])

    @pl.when(p == 1)
    def _():
        mean_s[rows, :] = jax.nn.relu(jnp.dot(adj_ref[...], t2_s[...]))

    @pl.when(p == 2)
    def _():
        l = jax.lax.dot_general(mean_s[rows, :], mean_s[...],
                                (((1,), (1,)), ((), ())))
        logits_ref[...] = l
        tile_max = jnp.max(l).reshape(1, 1)

        @pl.when(i == 0)
        def _():
            mx_s[...] = tile_max

        @pl.when(i != 0)
        def _():
            mx_s[...] = jnp.maximum(mx_s[...], tile_max)

    @pl.when(p == 3)
    def _():
        a = _adj_new_tile(mean_s[rows, :], mean_s[...], mx_s[...], i * _TM)
        d_s[rows, :] = jnp.power(jnp.sum(a, axis=1, keepdims=True), -0.5)

    @pl.when(p == 4)
    def _():
        a = _adj_new_tile(mean_s[rows, :], mean_s[...], mx_s[...], i * _TM)
        xd = d_s[...] * f2_s[...]
        h = jax.nn.relu(d_s[rows, :] * jnp.dot(a, xd) + b0_ref[...])
        v_s[rows, :] = d_s[rows, :] * jnp.dot(h, wn1_ref[...])

    @pl.when(p == 5)
    def _():
        a = _adj_new_tile(mean_s[rows, :], mean_s[...], mx_s[...], i * _TM)
        nc_ref[rows, :] = (d_s[rows, :] * jnp.dot(a, v_s[...])
                           + b1_ref[...])


def _const(shape):
    return pl.BlockSpec(shape, lambda p, i: (0,) * len(shape))


def kernel(adj, adj_orig, features, W_base, W_mean, W_nc0, b_nc0, W_nc1, b_nc1):
    del adj_orig
    f32 = jnp.float32
    b0 = b_nc0.reshape(1, _H)
    b1 = b_nc1.reshape(1, _C)

    adj_spec = pl.BlockSpec((_TM, _N),
                            lambda p, i: (jnp.where(p <= 1, i, _NT - 1), 0))
    logits_spec = pl.BlockSpec(
        (_TM, _N),
        lambda p, i: (jnp.where(p < 2, 0, jnp.where(p == 2, i, _NT - 1)), 0))

    nco, adj_logits = pl.pallas_call(
        _mega_kernel,
        grid=(6, _NT),
        in_specs=[adj_spec, _const((_N, 128)), _const((128, _H)),
                  _const((_H, _Z)), _const((128, _H)), _const((1, _H)),
                  _const((_H, _C)), _const((1, _C))],
        out_specs=[_const((_N, _C)), logits_spec],
        out_shape=[jax.ShapeDtypeStruct((_N, _C), f32),
                   jax.ShapeDtypeStruct((_N, _N), f32)],
        scratch_shapes=[pltpu.VMEM((_N, _H), f32), pltpu.VMEM((_N, _H), f32),
                        pltpu.VMEM((_N, _Z), f32), pltpu.VMEM((_N, _Z), f32),
                        pltpu.VMEM((1, 1), f32), pltpu.VMEM((_N, 1), f32),
                        pltpu.VMEM((_N, _C), f32)],
        compiler_params=pltpu.CompilerParams(
            vmem_limit_bytes=60 * 1024 * 1024),
    )(adj, features, W_base, W_mean, W_nc0, b0, W_nc1, b1)

    return (nco, adj_logits)


# speedup vs baseline: 1.2428x; 1.1470x over previous
import jax
import jax.numpy as jnp
from jax.experimental import pallas as pl
from jax.experimental.pallas import tpu as pltpu

_N = 4096
_TM = 512
_NT = _N // _TM
_H = 32
_Z = 16
_C = 7


def _adj_new_tile(mean_tile, mean_full, max_val, row_base):
    l = jax.lax.dot_general(mean_tile, mean_full, (((1,), (1,)), ((), ())))
    return _sample_tile(l, max_val, row_base)


def _sample_tile(l, max_val, row_base):
    s = jnp.round(l / max_val)
    rows = jax.lax.broadcasted_iota(jnp.int32, s.shape, 0) + row_base
    cols = jax.lax.broadcasted_iota(jnp.int32, s.shape, 1)
    return jnp.where(rows == cols, 1.0, s)


def _mega_kernel(adj_ref, feats_ref, wb_ref, wm_ref, wn0_ref, b0_ref,
                 wn1_ref, b1_ref,
                 nc_ref, logits_ref,
                 fb_s, f2_s, t2_s, mean_s, mx_s, d_s, v_s):
    p = pl.program_id(0)
    i = pl.program_id(1)
    rows = pl.ds(i * _TM, _TM)

    @pl.when((p == 0) & (i == 0))
    def _():
        f = feats_ref[...]
        fb_s[...] = jnp.dot(f, wb_ref[...])
        f2_s[...] = jnp.dot(f, wn0_ref[...])

    @pl.when(p == 0)
    def _():
        t2_s[rows, :] = jnp.dot(jnp.dot(adj_ref[...], fb_s[...]),
                                wm_ref[...])

    @pl.when(p == 1)
    def _():
        m = jax.nn.relu(jnp.dot(adj_ref[...], t2_s[...]))
        mean_s[rows, :] = m
        tile_max = jnp.max(jnp.sum(m * m, axis=1)).reshape(1, 1)

        @pl.when(i == 0)
        def _():
            mx_s[...] = tile_max

        @pl.when(i != 0)
        def _():
            mx_s[...] = jnp.maximum(mx_s[...], tile_max)

    @pl.when(p == 2)
    def _():
        l = jax.lax.dot_general(mean_s[rows, :], mean_s[...],
                                (((1,), (1,)), ((), ())))
        logits_ref[...] = l
        a = _sample_tile(l, mx_s[...], i * _TM)
        d_s[rows, :] = jnp.power(jnp.sum(a, axis=1, keepdims=True), -0.5)

    @pl.when(p == 3)
    def _():
        a = _adj_new_tile(mean_s[rows, :], mean_s[...], mx_s[...], i * _TM)
        xd = d_s[...] * f2_s[...]
        h = jax.nn.relu(d_s[rows, :] * jnp.dot(a, xd) + b0_ref[...])
        v_s[rows, :] = d_s[rows, :] * jnp.dot(h, wn1_ref[...])

    @pl.when(p == 4)
    def _():
        a = _adj_new_tile(mean_s[rows, :], mean_s[...], mx_s[...], i * _TM)
        nc_ref[rows, :] = (d_s[rows, :] * jnp.dot(a, v_s[...])
                           + b1_ref[...])


def _const(shape):
    return pl.BlockSpec(shape, lambda p, i: (0,) * len(shape))


def kernel(adj, adj_orig, features, W_base, W_mean, W_nc0, b_nc0, W_nc1, b_nc1):
    del adj_orig
    f32 = jnp.float32
    b0 = b_nc0.reshape(1, _H)
    b1 = b_nc1.reshape(1, _C)

    adj_spec = pl.BlockSpec((_TM, _N),
                            lambda p, i: (jnp.where(p <= 1, i, _NT - 1), 0))
    logits_spec = pl.BlockSpec(
        (_TM, _N),
        lambda p, i: (jnp.where(p < 2, 0, jnp.where(p == 2, i, _NT - 1)), 0))

    nco, adj_logits = pl.pallas_call(
        _mega_kernel,
        grid=(5, _NT),
        in_specs=[adj_spec, _const((_N, 128)), _const((128, _H)),
                  _const((_H, _Z)), _const((128, _H)), _const((1, _H)),
                  _const((_H, _C)), _const((1, _C))],
        out_specs=[_const((_N, _C)), logits_spec],
        out_shape=[jax.ShapeDtypeStruct((_N, _C), f32),
                   jax.ShapeDtypeStruct((_N, _N), f32)],
        scratch_shapes=[pltpu.VMEM((_N, _H), f32), pltpu.VMEM((_N, _H), f32),
                        pltpu.VMEM((_N, _Z), f32), pltpu.VMEM((_N, _Z), f32),
                        pltpu.VMEM((1, 1), f32), pltpu.VMEM((_N, 1), f32),
                        pltpu.VMEM((_N, _C), f32)],
        compiler_params=pltpu.CompilerParams(
            vmem_limit_bytes=60 * 1024 * 1024),
    )(adj, features, W_base, W_mean, W_nc0, b0, W_nc1, b1)

    return (nco, adj_logits)
